# baseline (speedup 1.0000x reference)
import functools

import jax
import jax.numpy as jnp
from jax import lax
from jax.experimental import pallas as pl
from jax.experimental.pallas import tpu as pltpu
from jax.experimental.pallas import tpu_sc as plsc

N_WORDS = 8192
K_ANT = 64
EMB = 32
N_PAIRS = N_WORDS * K_ANT

NUM_CORES = 2
NUM_SUBCORES = 16
NUM_TILES = NUM_CORES * NUM_SUBCORES
WORDS_PER_TILE = N_WORDS // NUM_TILES
LANES = 16
CHUNKS = WORDS_PER_TILE // LANES

N_CLASSES = 18


def _sc_classes_body(ti_hbm, spk_hbm, c_hbm, spk_v, ti_v, c_v):
    wid = lax.axis_index("s") * NUM_CORES + lax.axis_index("c")
    base_w = wid * WORDS_PER_TILE

    pltpu.sync_copy(spk_hbm, spk_v)
    pltpu.sync_copy(ti_hbm.at[:, pl.ds(base_w, WORDS_PER_TILE)], ti_v)

    iota16 = lax.iota(jnp.int32, LANES)

    def chunk_body(t, carry):
        off = t * LANES
        spk_i = spk_v[pl.ds(base_w + off, LANES)]
        i_vec = iota16 + (base_w + off)
        for j in range(K_ANT):
            ant = ti_v[j, pl.ds(off, LANES)]
            spk_a = plsc.load_gather(spk_v, [ant])
            same = (spk_a == spk_i).astype(jnp.int32)
            dist = jnp.maximum(i_vec - ant, 1)
            lg = (plsc.bitcast(dist.astype(jnp.float32), jnp.int32) >> 23) - 127
            didx = jnp.where(dist < 5, dist - 1, jnp.minimum(lg, 6) + 2)
            c_v[j, pl.ds(off, LANES)] = same * 9 + didx
        return carry

    lax.fori_loop(0, CHUNKS, chunk_body, 0)
    pltpu.sync_copy(c_v, c_hbm.at[:, pl.ds(base_w, WORDS_PER_TILE)])


def _sc_classes(ti_t, spk):
    sc = functools.partial(
        pl.kernel,
        out_type=jax.ShapeDtypeStruct((K_ANT, N_WORDS), jnp.int32),
        mesh=plsc.VectorSubcoreMesh(core_axis_name="c", subcore_axis_name="s"),
        scratch_types=[
            pltpu.VMEM((N_WORDS,), jnp.int32),
            pltpu.VMEM((K_ANT, WORDS_PER_TILE), jnp.int32),
            pltpu.VMEM((K_ANT, WORDS_PER_TILE), jnp.int32),
        ],
        compiler_params=pltpu.CompilerParams(needs_layout_passes=False),
    )(_sc_classes_body)
    return sc(ti_t, spk)


J_BLK = 8


def _tc_expand_body(c_ref, tblT_ref, o_ref):
    cb = c_ref[...].astype(jnp.int16)
    iota = lax.broadcasted_iota(jnp.int16, (N_CLASSES, N_WORDS), 0)
    tT = tblT_ref[...]
    for j in range(J_BLK):
        row = lax.broadcast_in_dim(cb[j], (N_CLASSES, N_WORDS), (1,))
        m = jnp.where(row == iota, jnp.bfloat16(1), jnp.bfloat16(0))
        o_ref[j * 96:(j + 1) * 96, :] = lax.dot_general(
            tT, m, (((1,), (0,)), ((), ())),
            preferred_element_type=jnp.float32)


def _tc_expand(c_t, tableT):
    return pl.pallas_call(
        _tc_expand_body,
        grid=(K_ANT // J_BLK,),
        in_specs=[
            pl.BlockSpec((J_BLK, N_WORDS), lambda i: (i, 0)),
            pl.BlockSpec((3 * EMB, N_CLASSES), lambda i: (0, 0)),
        ],
        compiler_params=pltpu.CompilerParams(
            dimension_semantics=("arbitrary",),
        ),
        out_specs=pl.BlockSpec((J_BLK * 3 * EMB, N_WORDS), lambda i: (i, 0)),
        out_shape=jax.ShapeDtypeStruct((K_ANT * 3 * EMB, N_WORDS), jnp.float32),
    )(c_t, tableT)


def kernel(top_indices, speaker_map, genre_id, genre_emb, distance_emb, speaker_emb):
    ti_t = jnp.swapaxes(top_indices.astype(jnp.int32), 0, 1)
    spk = speaker_map.astype(jnp.int32)

    c_t = _sc_classes(ti_t, spk)

    genre_row = jnp.take(genre_emb, jnp.asarray(genre_id, jnp.int32)[None], axis=0)
    table = jnp.concatenate(
        [
            jnp.repeat(speaker_emb, 9, axis=0),
            jnp.tile(distance_emb, (2, 1)),
            jnp.broadcast_to(genre_row, (N_CLASSES, EMB)),
        ],
        axis=1,
    )

    out2 = _tc_expand(c_t, table.T.astype(jnp.bfloat16))
    return out2.reshape(K_ANT, 3 * EMB, N_WORDS).transpose(2, 0, 1)


# speedup vs baseline: 1.0495x; 1.0495x over previous
import functools

import jax
import jax.numpy as jnp
from jax import lax
from jax.experimental import pallas as pl
from jax.experimental.pallas import tpu as pltpu
from jax.experimental.pallas import tpu_sc as plsc

N_WORDS = 8192
K_ANT = 64
EMB = 32
N_PAIRS = N_WORDS * K_ANT

NUM_CORES = 2
NUM_SUBCORES = 16
NUM_TILES = NUM_CORES * NUM_SUBCORES
WORDS_PER_TILE = N_WORDS // NUM_TILES
LANES = 16
CHUNKS = WORDS_PER_TILE // LANES

N_CLASSES = 18


def _sc_classes_body(ti_hbm, spk_hbm, c_hbm, spk_v, ti_v, c_v):
    wid = lax.axis_index("s") * NUM_CORES + lax.axis_index("c")
    base_w = wid * WORDS_PER_TILE

    pltpu.sync_copy(spk_hbm, spk_v)
    pltpu.sync_copy(ti_hbm.at[:, pl.ds(base_w, WORDS_PER_TILE)], ti_v)

    iota16 = lax.iota(jnp.int32, LANES)

    @plsc.parallel_loop(0, CHUNKS, step=1, unroll=2)
    def chunk_body(t):
        off = t * LANES
        spk_i = spk_v[pl.ds(base_w + off, LANES)]
        i_vec = iota16 + (base_w + off)
        for j in range(K_ANT):
            ant = ti_v[j, pl.ds(off, LANES)]
            spk_a = plsc.load_gather(spk_v, [ant])
            dist = jnp.maximum(i_vec - ant, 1)
            lg = (plsc.bitcast(dist.astype(jnp.float32), jnp.int32) >> 23) - 127
            didx = jnp.where(dist < 5, dist - 1, jnp.minimum(lg, 6) + 2)
            c_v[j, pl.ds(off, LANES)] = didx + jnp.where(spk_a == spk_i, 9, 0)
    pltpu.sync_copy(c_v, c_hbm.at[:, pl.ds(base_w, WORDS_PER_TILE)])


def _sc_classes(ti_t, spk):
    sc = functools.partial(
        pl.kernel,
        out_type=jax.ShapeDtypeStruct((K_ANT, N_WORDS), jnp.int32),
        mesh=plsc.VectorSubcoreMesh(core_axis_name="c", subcore_axis_name="s"),
        scratch_types=[
            pltpu.VMEM((N_WORDS,), jnp.int32),
            pltpu.VMEM((K_ANT, WORDS_PER_TILE), jnp.int32),
            pltpu.VMEM((K_ANT, WORDS_PER_TILE), jnp.int32),
        ],
        compiler_params=pltpu.CompilerParams(needs_layout_passes=False),
    )(_sc_classes_body)
    return sc(ti_t, spk)


WBLK = 512


def _tc_expand_body(c_ref, tblT_ref, o_ref):
    cb = c_ref[...].astype(jnp.int16)
    iota = lax.broadcasted_iota(jnp.int16, (N_CLASSES, WBLK), 0)
    tT = tblT_ref[...]
    for j in range(K_ANT):
        row = lax.broadcast_in_dim(cb[j], (N_CLASSES, WBLK), (1,))
        m = jnp.where(row == iota, jnp.bfloat16(1), jnp.bfloat16(0))
        o_ref[j * 96:(j + 1) * 96, :] = lax.dot_general(
            tT, m, (((1,), (0,)), ((), ())),
            preferred_element_type=jnp.float32)


def _tc_expand(c_t, tableT):
    return pl.pallas_call(
        _tc_expand_body,
        grid=(N_WORDS // WBLK,),
        in_specs=[
            pl.BlockSpec((K_ANT, WBLK), lambda i: (0, i)),
            pl.BlockSpec((3 * EMB, N_CLASSES), lambda i: (0, 0)),
        ],
        compiler_params=pltpu.CompilerParams(
            dimension_semantics=("arbitrary",),
        ),
        out_specs=pl.BlockSpec((K_ANT * 3 * EMB, WBLK), lambda i: (0, i)),
        out_shape=jax.ShapeDtypeStruct((K_ANT * 3 * EMB, N_WORDS), jnp.float32),
    )(c_t, tableT)


def kernel(top_indices, speaker_map, genre_id, genre_emb, distance_emb, speaker_emb):
    ti_t = jnp.swapaxes(top_indices.astype(jnp.int32), 0, 1)
    spk = speaker_map.astype(jnp.int32)

    c_t = _sc_classes(ti_t, spk)

    genre_row = jnp.take(genre_emb, jnp.asarray(genre_id, jnp.int32)[None], axis=0)
    table = jnp.concatenate(
        [
            jnp.repeat(speaker_emb, 9, axis=0),
            jnp.tile(distance_emb, (2, 1)),
            jnp.broadcast_to(genre_row, (N_CLASSES, EMB)),
        ],
        axis=1,
    )

    out2 = _tc_expand(c_t, table.T.astype(jnp.bfloat16))
    return out2.reshape(K_ANT, 3 * EMB, N_WORDS).transpose(2, 0, 1)
